# Initial kernel scaffold; baseline (speedup 1.0000x reference)
import jax
import jax.numpy as jnp
from jax.experimental import pallas as pl


def kernel(x, gamma1, gamma2, beta1, beta2, r, edge_index, node_degrees, node_types):
    raise NotImplementedError("write your pallas kernel here")


# speedup vs baseline: 7.4665x; 7.4665x over previous
import functools

import jax
import jax.numpy as jnp
from jax import lax
from jax.experimental import pallas as pl
from jax.experimental.pallas import tpu as pltpu
from jax.experimental.pallas import tpu_sc as plsc

N = 10000
E = 320000
D = 128
HD = D // 2
TAIL_K = 5

NC = 2
NS = 16

CHUNK = 128
_rpt = (E + NS * CHUNK - 1) // (NS * CHUNK)
RPT = (_rpt + 7) // 8 * 8
EP = NS * RPT * CHUNK
EROWS = EP // CHUNK

NP = 10112
STRIPE = NP // NS
NTRASH = NP - 1

_mesh = plsc.VectorSubcoreMesh(core_axis_name="c", subcore_axis_name="s")
_f32 = jnp.float32
_i32 = jnp.int32


def _zero_rows(ref):
    def body(i, _):
        for k in range(HD // 16):
            ref[i, pl.ds(k * 16, 16)] = jnp.zeros((16,), _f32)
        return jnp.int32(0)
    lax.fori_loop(jnp.int32(0), jnp.int32(128), body, jnp.int32(0))


def _fill_rows16(ref, val):
    def body(i, _):
        ref[i, pl.ds(0, 16)] = jnp.full((16,), val, _f32)
        return jnp.int32(0)
    lax.fori_loop(jnp.int32(0), jnp.int32(128), body, jnp.int32(0))


def _zero_stripe(shared, sb, zbuf):
    nfull = STRIPE // 128
    rem = STRIPE - nfull * 128
    for t in range(nfull):
        pltpu.sync_copy(zbuf, shared.at[pl.ds(sb + t * 128, 128)])
    if rem:
        pltpu.sync_copy(zbuf.at[pl.ds(0, rem)], shared.at[pl.ds(sb + nfull * 128, rem)])


@functools.partial(
    pl.kernel,
    mesh=_mesh,
    out_type=(
        jax.ShapeDtypeStruct((NC, NP, HD), _f32),
        jax.ShapeDtypeStruct((NP, 16), _f32),
    ),
    compiler_params=pltpu.CompilerParams(use_tc_tiling_on_sc=False,
                                         needs_layout_passes=False),
    scratch_types=[
        pltpu.VMEM((RPT, CHUNK), _i32),
        pltpu.VMEM((RPT, CHUNK), _i32),
        pltpu.VMEM((CHUNK, HD), _f32),
        pltpu.VMEM((CHUNK, 16), _f32),
        pltpu.VMEM((CHUNK, 16), _f32),
        pltpu.VMEM_SHARED((NP, HD), _f32),
        pltpu.VMEM_SHARED((NP, 16), _f32),
        pltpu.SemaphoreType.DMA,
    ],
)
def _sc_layer1(src_hbm, dst_hbm, x_hbm, raw_out, deg_out,
               src_v, dst_v, rows_v, ones_v, zc_v, acc_sh, dacc_sh, sem):
    c = lax.axis_index("c")
    s = lax.axis_index("s")
    sb = s * STRIPE

    _zero_rows(rows_v)
    _fill_rows16(ones_v, 1.0)
    _fill_rows16(zc_v, 0.0)
    _zero_stripe(acc_sh, sb, rows_v)

    @pl.when(c == 0)
    def _():
        _zero_stripe(dacc_sh, sb, zc_v)

    pltpu.sync_copy(src_hbm.at[pl.ds(s * RPT, RPT)], src_v)
    pltpu.sync_copy(dst_hbm.at[pl.ds(s * RPT, RPT)], dst_v)

    ntr = jnp.full((16,), NTRASH, _i32)
    two16 = jnp.full((16,), 2, _i32)
    cvec = jnp.broadcast_to(c, (16,)).astype(_i32)

    def idx_body(g, _):
        for k in range(CHUNK // 16):
            sl = pl.ds(k * 16, 16)
            sv = src_v[g, sl]
            dv = dst_v[g, sl]
            src_v[g, sl] = jnp.where(sv == dv, ntr, sv)
            dst_v[g, sl] = dv * two16 + cvec
        return jnp.int32(0)
    lax.fori_loop(jnp.int32(0), jnp.int32(RPT), idx_body, jnp.int32(0))

    plsc.subcore_barrier()

    def edge_body(j, _):
        pltpu.async_copy(x_hbm.at[dst_v.at[j]], rows_v, sem).wait()
        pltpu.sync_copy(rows_v, acc_sh.at[src_v.at[j]], add=True)

        @pl.when(c == 0)
        def _():
            pltpu.sync_copy(ones_v, dacc_sh.at[src_v.at[j]], add=True)
        return jnp.int32(0)
    lax.fori_loop(jnp.int32(0), jnp.int32(RPT), edge_body, jnp.int32(0))

    plsc.subcore_barrier()

    pltpu.sync_copy(acc_sh.at[pl.ds(sb, STRIPE)], raw_out.at[c, pl.ds(sb, STRIPE)])

    @pl.when(c == 0)
    def _():
        pltpu.sync_copy(dacc_sh.at[pl.ds(sb, STRIPE)], deg_out.at[pl.ds(sb, STRIPE)])


@functools.partial(
    pl.kernel,
    mesh=_mesh,
    out_type=jax.ShapeDtypeStruct((NC, NP, HD), _f32),
    compiler_params=pltpu.CompilerParams(needs_layout_passes=False,
                                         use_tc_tiling_on_sc=False),
    scratch_types=[
        pltpu.VMEM((RPT, CHUNK), _i32),
        pltpu.VMEM((RPT, CHUNK), _i32),
        pltpu.VMEM((CHUNK, HD), _f32),
        pltpu.VMEM((N,), _i32),
        pltpu.VMEM((N,), _i32),
        pltpu.VMEM((N,), _i32),
        pltpu.VMEM_SHARED((NP, HD), _f32),
        pltpu.SemaphoreType.DMA,
    ],
)
def _sc_layer2(src_hbm, dst_hbm, tab_hbm, nd_hbm, nt_hbm, raw_out,
               src_v, dst_v, rows_v, nd_v, nt_v, hm_v, acc_sh, sem):
    c = lax.axis_index("c")
    s = lax.axis_index("s")
    sb = s * STRIPE

    _zero_rows(rows_v)
    _zero_stripe(acc_sh, sb, rows_v)

    pltpu.sync_copy(src_hbm.at[pl.ds(s * RPT, RPT)], src_v)
    pltpu.sync_copy(dst_hbm.at[pl.ds(s * RPT, RPT)], dst_v)
    pltpu.sync_copy(nd_hbm, nd_v)
    pltpu.sync_copy(nt_hbm, nt_v)

    tailk = jnp.full((16,), TAIL_K, _i32)
    zeros16 = jnp.full((16,), 0, _i32)
    ones16 = jnp.full((16,), 1, _i32)
    n2vec = jnp.full((16,), 2 * N, _i32)
    two16 = jnp.full((16,), 2, _i32)
    ntr = jnp.full((16,), NTRASH, _i32)
    cvec = jnp.broadcast_to(c, (16,)).astype(_i32)

    def hm_body(i, _):
        o = i * jnp.int32(16)
        ndv = nd_v[pl.ds(o, 16)]
        ntv = nt_v[pl.ds(o, 16)]
        hm_v[pl.ds(o, 16)] = jnp.where((ndv > tailk) | (ntv != zeros16), ones16, zeros16)
        return jnp.int32(0)
    lax.fori_loop(jnp.int32(0), jnp.int32(N // 16), hm_body, jnp.int32(0))

    def idx_body(g, _):
        for k in range(CHUNK // 16):
            sl = pl.ds(k * 16, 16)
            sv = src_v[g, sl]
            dv = dst_v[g, sl]
            hs = plsc.load_gather(hm_v, [sv])
            dst_v[g, sl] = dv * two16 + (ones16 - hs) * n2vec + cvec
            src_v[g, sl] = jnp.where(sv == dv, ntr, sv)
        return jnp.int32(0)
    lax.fori_loop(jnp.int32(0), jnp.int32(RPT), idx_body, jnp.int32(0))

    plsc.subcore_barrier()

    def edge_body(j, _):
        pltpu.async_copy(tab_hbm.at[dst_v.at[j]], rows_v, sem).wait()
        pltpu.sync_copy(rows_v, acc_sh.at[src_v.at[j]], add=True)
        return jnp.int32(0)
    lax.fori_loop(jnp.int32(0), jnp.int32(RPT), edge_body, jnp.int32(0))

    plsc.subcore_barrier()

    pltpu.sync_copy(acc_sh.at[pl.ds(sb, STRIPE)], raw_out.at[c, pl.ds(sb, STRIPE)])


ROWB = 2000


def _mm(a, w):
    return lax.dot_general(a, w, (((1,), (1,)), ((), ())),
                           preferred_element_type=_f32)


def _lrelu(z):
    return jnp.maximum(z, 0.2 * z)


def _tc1_body(x_ref, rawp_ref, deg_ref, nd_ref, nt_ref,
              g1_ref, g2_ref, b1_ref, b2_ref, r_ref, tab_ref):
    xv = x_ref[...]
    raw1 = jnp.concatenate([rawp_ref[0], rawp_ref[1]], axis=1)
    deg = deg_ref[:, 0:1]
    inv = 1.0 / jnp.maximum(deg, 1.0)
    d1 = 1.0 / (deg + 1.0)
    d2 = 1.0 / (deg + 2.0)
    n1 = raw1 * inv
    gz = _mm(xv, g1_ref[...]) + _mm(n1, g2_ref[...])
    bz = _mm(xv, b1_ref[...]) + _mm(n1, b2_ref[...])
    m1 = xv + (_lrelu(gz) + 1.0) * r_ref[...] + _lrelu(bz) - n1
    base = raw1 + xv
    tab_ref[0] = base * d1
    tab_ref[1] = (base + m1) * d2


def _tc2_body(x_ref, tab_ref, rawp2_ref, deg_ref, nd_ref, nt_ref,
              g1_ref, g2_ref, b1_ref, b2_ref, r_ref, out_ref):
    hmf = ((nd_ref[...] > TAIL_K) | (nt_ref[...] != 0)).astype(_f32)
    cur = tab_ref[0] * hmf + tab_ref[1] * (1.0 - hmf)
    raw2 = jnp.concatenate([rawp2_ref[0], rawp2_ref[1]], axis=1)
    deg = deg_ref[:, 0:1]
    inv = 1.0 / jnp.maximum(deg, 1.0)
    d1 = 1.0 / (deg + 1.0)
    d2 = 1.0 / (deg + 2.0)
    n2 = raw2 * inv
    gz = _mm(cur, g1_ref[...]) + _mm(n2, g2_ref[...])
    bz = _mm(cur, b1_ref[...]) + _mm(n2, b2_ref[...])
    m2 = cur + (_lrelu(gz) + 1.0) * r_ref[...] + _lrelu(bz) - n2
    base = raw2 + cur
    h2 = base * d1 * hmf + (base + m2) * d2 * (1.0 - hmf)
    out_ref[...] = (x_ref[...] + cur + h2) * (1.0 / 3.0)


def _zz():
    return jnp.int32(0)


_W_SPEC = pl.BlockSpec((D, D), lambda i: (_zz(), _zz()))
_R_SPEC = pl.BlockSpec((1, D), lambda i: (_zz(), _zz()))
_COL_SPEC = pl.BlockSpec((ROWB, 1), lambda i: (i, _zz()))
_PART_SPEC = pl.BlockSpec((NC, ROWB, HD), lambda i: (_zz(), i, _zz()))
_DEG_SPEC = pl.BlockSpec((ROWB, 16), lambda i: (i, _zz()))
_X_SPEC = pl.BlockSpec((ROWB, D), lambda i: (i, _zz()))
_TAB_SPEC = pl.BlockSpec((2, ROWB, D), lambda i: (_zz(), i, _zz()))

_tc1 = pl.pallas_call(
    _tc1_body,
    grid=(N // ROWB,),
    in_specs=[_X_SPEC, _PART_SPEC, _DEG_SPEC, _COL_SPEC, _COL_SPEC,
              _W_SPEC, _W_SPEC, _W_SPEC, _W_SPEC, _R_SPEC],
    out_specs=_TAB_SPEC,
    out_shape=jax.ShapeDtypeStruct((2, N, D), _f32),
)

_tc2 = pl.pallas_call(
    _tc2_body,
    grid=(N // ROWB,),
    in_specs=[_X_SPEC, _TAB_SPEC, _PART_SPEC, _DEG_SPEC, _COL_SPEC, _COL_SPEC,
              _W_SPEC, _W_SPEC, _W_SPEC, _W_SPEC, _R_SPEC],
    out_specs=_X_SPEC,
    out_shape=jax.ShapeDtypeStruct((N, D), _f32),
)


def kernel(x, gamma1, gamma2, beta1, beta2, r, edge_index, node_degrees, node_types):
    ei = edge_index.astype(_i32)
    src = jnp.pad(ei[0], (0, EP - E)).reshape(EROWS, CHUNK)
    dst = jnp.pad(ei[1], (0, EP - E)).reshape(EROWS, CHUNK)
    nd32 = node_degrees.astype(_i32)
    nt32 = node_types.astype(_i32)
    nd_col = nd32.reshape(N, 1)
    nt_col = nt32.reshape(N, 1)
    x = x.astype(_f32)

    rawp, deg = _sc_layer1(src, dst, x.reshape(2 * N, HD))
    tab = _tc1(x, rawp, deg, nd_col, nt_col,
               gamma1[0], gamma2[0], beta1[0], beta2[0], r[0])
    rawp2 = _sc_layer2(src, dst, tab.reshape(4 * N, HD), nd32, nt32)
    out = _tc2(x, tab, rawp2, deg, nd_col, nt_col,
               gamma1[1], gamma2[1], beta1[1], beta2[1], r[1])
    return out
